# baseline (device time: 35484 ns/iter reference)
import jax
import jax.numpy as jnp
from jax import lax
from jax.experimental import pallas as pl
from jax.experimental.pallas import tpu as pltpu


def kernel(O, Wo):
    B, S, H, D = O.shape
    K = H * D
    N = Wo.shape[1]
    S_half = S // 2

    O2 = O.reshape(B, S, K).astype(jnp.bfloat16)
    Wo_b = Wo.astype(jnp.bfloat16)

    def body(o_ref, wo_ref, out_ref, send_buf, recv_buf, send_sem, recv_sem):
        my_x = lax.axis_index("x")
        my_y = lax.axis_index("y")
        my_z = lax.axis_index("z")
        partner = (my_x, my_y, 1 - my_z)

        barrier_sem = pltpu.get_barrier_semaphore()
        pl.semaphore_signal(
            barrier_sem, inc=1,
            device_id=partner, device_id_type=pl.DeviceIdType.MESH,
        )
        pl.semaphore_wait(barrier_sem, 1)

        wo = wo_ref[...]
        p0 = (1 - my_z) * S_half
        m0 = my_z * S_half

        for b in range(B):
            acc = jnp.dot(
                o_ref[b, pl.ds(p0, S_half), :], wo,
                preferred_element_type=jnp.float32,
            )
            send_buf[b] = acc.astype(jnp.bfloat16)

        rdma = pltpu.make_async_remote_copy(
            src_ref=send_buf,
            dst_ref=recv_buf,
            send_sem=send_sem,
            recv_sem=recv_sem,
            device_id=partner,
            device_id_type=pl.DeviceIdType.MESH,
        )
        rdma.start()

        for b in range(B):
            out_ref[b] = jnp.dot(
                o_ref[b, pl.ds(m0, S_half), :], wo,
                preferred_element_type=jnp.float32,
            )

        rdma.wait()
        for b in range(B):
            out_ref[b] = out_ref[b] + recv_buf[b].astype(jnp.float32)

    out_shape = jax.ShapeDtypeStruct((B, S_half, N), jnp.float32)
    return pl.pallas_call(
        body,
        out_shape=out_shape,
        in_specs=[
            pl.BlockSpec(memory_space=pltpu.VMEM),
            pl.BlockSpec(memory_space=pltpu.VMEM),
        ],
        out_specs=pl.BlockSpec(memory_space=pltpu.VMEM),
        scratch_shapes=[
            pltpu.VMEM((B, S_half, N), jnp.bfloat16),
            pltpu.VMEM((B, S_half, N), jnp.bfloat16),
            pltpu.SemaphoreType.DMA,
            pltpu.SemaphoreType.DMA,
        ],
        compiler_params=pltpu.CompilerParams(collective_id=0),
    )(O2, Wo_b)


# device time: 32881 ns/iter; 1.0792x vs baseline; 1.0792x over previous
import jax
import jax.numpy as jnp
from jax import lax
from jax.experimental import pallas as pl
from jax.experimental.pallas import tpu as pltpu


def kernel(O, Wo):
    B, S, H, D = O.shape
    K = H * D
    N = Wo.shape[1]
    S_half = S // 2

    O2 = O.reshape(B, S, K)

    def body(o_ref, wo_ref, out_ref, send_buf, recv_buf, send_sems, recv_sems):
        my_x = lax.axis_index("x")
        my_y = lax.axis_index("y")
        my_z = lax.axis_index("z")
        partner = (my_x, my_y, 1 - my_z)

        barrier_sem = pltpu.get_barrier_semaphore()
        pl.semaphore_signal(
            barrier_sem, inc=1,
            device_id=partner, device_id_type=pl.DeviceIdType.MESH,
        )
        pl.semaphore_wait(barrier_sem, 1)

        wo = wo_ref[...].astype(jnp.bfloat16)
        p0 = (1 - my_z) * S_half
        m0 = my_z * S_half

        rdmas = []
        for b in range(B):
            acc = jnp.dot(
                o_ref[b, pl.ds(p0, S_half), :].astype(jnp.bfloat16), wo,
                preferred_element_type=jnp.float32,
            )
            send_buf[b] = acc.astype(jnp.bfloat16)
            r = pltpu.make_async_remote_copy(
                src_ref=send_buf.at[b],
                dst_ref=recv_buf.at[b],
                send_sem=send_sems.at[b],
                recv_sem=recv_sems.at[b],
                device_id=partner,
                device_id_type=pl.DeviceIdType.MESH,
            )
            r.start()
            rdmas.append(r)

        for b in range(B):
            out_ref[b] = jnp.dot(
                o_ref[b, pl.ds(m0, S_half), :].astype(jnp.bfloat16), wo,
                preferred_element_type=jnp.float32,
            )

        for b in range(B):
            rdmas[b].wait_recv()
            out_ref[b] = out_ref[b] + recv_buf[b].astype(jnp.float32)
        for b in range(B):
            rdmas[b].wait_send()

    out_shape = jax.ShapeDtypeStruct((B, S_half, N), jnp.float32)
    return pl.pallas_call(
        body,
        out_shape=out_shape,
        in_specs=[
            pl.BlockSpec(memory_space=pltpu.VMEM),
            pl.BlockSpec(memory_space=pltpu.VMEM),
        ],
        out_specs=pl.BlockSpec(memory_space=pltpu.VMEM),
        scratch_shapes=[
            pltpu.VMEM((B, S_half, N), jnp.bfloat16),
            pltpu.VMEM((B, S_half, N), jnp.bfloat16),
            pltpu.SemaphoreType.DMA((B,)),
            pltpu.SemaphoreType.DMA((B,)),
        ],
        compiler_params=pltpu.CompilerParams(collective_id=0),
    )(O2, Wo)
